# baseline (speedup 1.0000x reference)
import functools

import jax
import jax.numpy as jnp
from jax import lax
from jax.experimental import pallas as pl
from jax.experimental.pallas import tpu as pltpu
from jax.experimental.pallas import tpu_sc as plsc

N_LAT = 32
N_OUT = 32
LANES = 16
SB = 512


def _pack_body(in_ref, out_ref):
    x = in_ref[...]
    u16 = jax.lax.bitcast_convert_type(
        x.astype(jnp.bfloat16), jnp.uint16).astype(jnp.int32)
    r = u16.reshape(N_LAT, LANES, 2, SB)
    w = r[:, :, 0, :] | (r[:, :, 1, :] << 16)
    wt = jnp.transpose(w.reshape(N_LAT * LANES, SB), (1, 0))
    out_ref[...] = wt.reshape(SB, 4, 128)


def _build_pack(size, blk_off):
    return pl.pallas_call(
        _pack_body,
        grid=(pl.cdiv(size, SB),),
        in_specs=[pl.BlockSpec((N_LAT, N_OUT, SB), lambda i: (0, 0, i + blk_off))],
        out_specs=pl.BlockSpec((SB, 4, 128), lambda i: (i, 0, 0)),
        out_shape=jax.ShapeDtypeStruct((size, 4, 128), jnp.int32),
    )


def _build_sc(B, N_SAMPLE, lo, size):
    info = plsc.get_sparse_core_info()
    NC, NS = info.num_cores, info.num_subcores
    NW = NC * NS
    assert B % NW == 0 and N_SAMPLE % 4 == 0
    RPW = B // NW
    C = 32
    NPAIR = RPW // (2 * C)

    mesh = plsc.VectorSubcoreMesh(core_axis_name="c", subcore_axis_name="s")

    @functools.partial(
        pl.kernel,
        mesh=mesh,
        out_type=jax.ShapeDtypeStruct((N_OUT, B), jnp.float32),
        compiler_params=pltpu.CompilerParams(needs_layout_passes=False),
        scratch_types=[
            pltpu.VMEM((RPW,), jnp.int32),
            pltpu.VMEM((RPW,), jnp.int32),
            pltpu.VMEM((RPW,), jnp.int32),
            pltpu.VMEM((C, 4, 128), jnp.int32),
            pltpu.VMEM((C, 4, 128), jnp.int32),
            pltpu.VMEM((C, 128), jnp.float32),
            pltpu.VMEM((C, 128), jnp.float32),
            pltpu.VMEM((N_LAT, RPW), jnp.float32),
            pltpu.VMEM((N_OUT, RPW), jnp.float32),
            pltpu.SemaphoreType.DMA,
            pltpu.SemaphoreType.DMA,
            pltpu.SemaphoreType.DMA,
            pltpu.SemaphoreType.DMA,
        ],
    )
    def k(u_t_hbm, si_hbm, amat_hbm, offs4_hbm, out_t_hbm,
          idx_v, idx4_v, idxr_v, as_p, as_q, og_p, og_q, u_slab, out_slab,
          sem_as_p, sem_as_q, sem_og_p, sem_og_q):
        wid = lax.axis_index("s") * NC + lax.axis_index("c")
        base = wid * RPW
        pltpu.sync_copy(si_hbm.at[pl.ds(base, RPW)], idx_v)
        pltpu.sync_copy(u_t_hbm.at[:, pl.ds(base, RPW)], u_slab)

        def shift(g, carry):
            sl = pl.ds(g * LANES, LANES)
            sv = idx_v[sl]
            idx4_v[sl] = lax.shift_right_logical(sv, 2)
            idxr_v[sl] = sv
            idx_v[sl] = jnp.clip(sv - lo, 0, size - 1)
            return carry

        lax.fori_loop(0, RPW // LANES, shift, 0)

        def start(off, as_b, og_b, sem_a, sem_o):
            pltpu.async_copy(amat_hbm.at[idx_v.at[pl.ds(off, C)]], as_b, sem_a)
            pltpu.async_copy(offs4_hbm.at[idx4_v.at[pl.ds(off, C)]], og_b, sem_o)

        def wait(as_b, og_b, sem_a, sem_o):
            pltpu.make_async_copy(amat_hbm.at[idx_v.at[pl.ds(0, C)]], as_b, sem_a).wait()
            pltpu.make_async_copy(offs4_hbm.at[idx4_v.at[pl.ds(0, C)]], og_b, sem_o).wait()

        def compute(off, as_b, og_b):
            def octet(ro, carry):
                rbase = ro * 8
                sv = plsc.load_gather(
                    idxr_v, [lax.iota(jnp.int32, LANES) + (off + rbase)])
                obase = (sv & 3) * N_OUT
                zf = jnp.where((sv >= lo) & (sv < lo + size), 1.0, 0.0)
                for rr in range(8):
                    r = rbase + rr
                    j = off + r
                    rows = lax.iota(jnp.int32, LANES)
                    evens = rows * 2
                    odds = evens + 1
                    cols = jnp.full((LANES,), j, jnp.int32)
                    zfr = zf[rr]
                    uv0 = plsc.load_gather(u_slab, [rows, cols]) * zfr
                    uv1 = plsc.load_gather(u_slab, [rows + LANES, cols]) * zfr
                    ob = obase[rr]
                    rvec = jnp.full((LANES,), r, jnp.int32)
                    og_e = plsc.load_gather(og_b, [rvec, evens + ob])
                    og_o = plsc.load_gather(og_b, [rvec, odds + ob])
                    u_e = plsc.load_gather(u_slab, [evens, cols])
                    u_o = plsc.load_gather(u_slab, [odds, cols])
                    acc_e = (u_e + og_e) * zfr
                    acc_o = (u_o + og_o) * zfr
                    for l in range(N_LAT):
                        ul = (uv0 if l < LANES else uv1)[l % LANES]
                        w = as_b[r, l // 8, pl.ds((l % 8) * LANES, LANES)]
                        ae = plsc.bitcast(lax.shift_left(w, 16), jnp.float32)
                        ao = plsc.bitcast(w & jnp.int32(-65536), jnp.float32)
                        acc_e = acc_e + ul * ae
                        acc_o = acc_o + ul * ao
                    plsc.store_scatter(out_slab, [evens, cols], acc_e)
                    plsc.store_scatter(out_slab, [odds, cols], acc_o)
                return carry

            lax.fori_loop(0, C // 8, octet, 0)

        start(0, as_p, og_p, sem_as_p, sem_og_p)

        def pair(i, carry):
            offp = (2 * i) * C
            offq = (2 * i + 1) * C
            start(offq, as_q, og_q, sem_as_q, sem_og_q)
            wait(as_p, og_p, sem_as_p, sem_og_p)
            compute(offp, as_p, og_p)

            @pl.when(i < NPAIR - 1)
            def _():
                start(offq + C, as_p, og_p, sem_as_p, sem_og_p)

            wait(as_q, og_q, sem_as_q, sem_og_q)
            compute(offq, as_q, og_q)
            return carry

        lax.fori_loop(0, NPAIR, pair, 0)
        pltpu.sync_copy(out_slab, out_t_hbm.at[:, pl.ds(base, RPW)])

    return k


def kernel(u, sample_index, amat_sample, offsets):
    B = u.shape[0]
    n_sample = amat_sample.shape[0]
    si = sample_index.squeeze() if sample_index.ndim > 1 else sample_index
    si = si.astype(jnp.int32)
    amat_t = amat_sample.transpose(1, 2, 0)
    offs4 = offsets.reshape(n_sample // 4, 128)
    u_t = u.T
    half = (n_sample // (2 * SB)) * SB
    sizes = [half, n_sample - half]
    lows = [0, half]
    out_t = None
    for lo, size in zip(lows, sizes):
        packed = _build_pack(size, lo // SB)(amat_t)
        part = _build_sc(B, n_sample, lo, size)(u_t, si, packed, offs4)
        out_t = part if out_t is None else out_t + part
    return out_t.T


# speedup vs baseline: 2.2925x; 2.2925x over previous
import functools

import jax
import jax.numpy as jnp
from jax import lax
from jax.experimental import pallas as pl
from jax.experimental.pallas import tpu as pltpu
from jax.experimental.pallas import tpu_sc as plsc

N_LAT = 32
N_OUT = 32
LANES = 16
SB = 512


def _pack_body(in_ref, out_ref):
    x = in_ref[...]
    u16 = jax.lax.bitcast_convert_type(
        x.astype(jnp.bfloat16), jnp.uint16).astype(jnp.int32)
    r = u16.reshape(N_LAT, LANES, 2, SB)
    w = r[:, :, 0, :] | (r[:, :, 1, :] << 16)
    wt = jnp.transpose(w.reshape(N_LAT * LANES, SB), (1, 0))
    out_ref[...] = wt.reshape(SB, 4, 128)


def _build_pack(size, blk_off):
    return pl.pallas_call(
        _pack_body,
        grid=(pl.cdiv(size, SB),),
        in_specs=[pl.BlockSpec((N_LAT, N_OUT, SB), lambda i: (0, 0, i + blk_off))],
        out_specs=pl.BlockSpec((SB, 4, 128), lambda i: (i, 0, 0)),
        out_shape=jax.ShapeDtypeStruct((size, 4, 128), jnp.int32),
    )


def _build_sc(B, N_SAMPLE, lo, size):
    info = plsc.get_sparse_core_info()
    NC, NS = info.num_cores, info.num_subcores
    NW = NC * NS
    assert B % NW == 0 and N_SAMPLE % 4 == 0
    RPW = B // NW
    C = 32
    NPAIR = RPW // (2 * C)

    mesh = plsc.VectorSubcoreMesh(core_axis_name="c", subcore_axis_name="s")

    @functools.partial(
        pl.kernel,
        mesh=mesh,
        out_type=jax.ShapeDtypeStruct((N_OUT, B), jnp.float32),
        compiler_params=pltpu.CompilerParams(needs_layout_passes=False),
        scratch_types=[
            pltpu.VMEM((RPW,), jnp.int32),
            pltpu.VMEM((RPW,), jnp.int32),
            pltpu.VMEM((RPW,), jnp.int32),
            pltpu.VMEM((C, 4, 128), jnp.int32),
            pltpu.VMEM((C, 4, 128), jnp.int32),
            pltpu.VMEM((C, 128), jnp.float32),
            pltpu.VMEM((C, 128), jnp.float32),
            pltpu.VMEM((N_LAT, RPW), jnp.float32),
            pltpu.VMEM((N_OUT, RPW), jnp.float32),
            pltpu.SemaphoreType.DMA,
            pltpu.SemaphoreType.DMA,
            pltpu.SemaphoreType.DMA,
            pltpu.SemaphoreType.DMA,
        ],
    )
    def k(u_t_hbm, si_hbm, amat_hbm, offs4_hbm, out_t_hbm,
          idx_v, idx4_v, idxr_v, as_p, as_q, og_p, og_q, u_slab, out_slab,
          sem_as_p, sem_as_q, sem_og_p, sem_og_q):
        wid = lax.axis_index("s") * NC + lax.axis_index("c")
        base = wid * RPW
        pltpu.sync_copy(si_hbm.at[pl.ds(base, RPW)], idx_v)
        pltpu.sync_copy(u_t_hbm.at[:, pl.ds(base, RPW)], u_slab)

        def shift(g, carry):
            sl = pl.ds(g * LANES, LANES)
            sv = idx_v[sl]
            idx4_v[sl] = lax.shift_right_logical(sv, 2)
            idxr_v[sl] = sv
            inr = (sv >= lo) & (sv < lo + size)
            idx_v[sl] = jnp.where(inr, sv - lo, sv & 4095)
            return carry

        lax.fori_loop(0, RPW // LANES, shift, 0)

        def start(off, as_b, og_b, sem_a, sem_o):
            pltpu.async_copy(amat_hbm.at[idx_v.at[pl.ds(off, C)]], as_b, sem_a)
            pltpu.async_copy(offs4_hbm.at[idx4_v.at[pl.ds(off, C)]], og_b, sem_o)

        def wait(as_b, og_b, sem_a, sem_o):
            pltpu.make_async_copy(amat_hbm.at[idx_v.at[pl.ds(0, C)]], as_b, sem_a).wait()
            pltpu.make_async_copy(offs4_hbm.at[idx4_v.at[pl.ds(0, C)]], og_b, sem_o).wait()

        def compute(off, as_b, og_b):
            def octet(ro, carry):
                rbase = ro * 8
                sv = plsc.load_gather(
                    idxr_v, [lax.iota(jnp.int32, LANES) + (off + rbase)])
                obase = (sv & 3) * N_OUT
                zf = jnp.where((sv >= lo) & (sv < lo + size), 1.0, 0.0)
                for rr in range(8):
                    r = rbase + rr
                    j = off + r
                    rows = lax.iota(jnp.int32, LANES)
                    evens = rows * 2
                    odds = evens + 1
                    cols = jnp.full((LANES,), j, jnp.int32)
                    zfr = zf[rr]
                    uv0 = plsc.load_gather(u_slab, [rows, cols]) * zfr
                    uv1 = plsc.load_gather(u_slab, [rows + LANES, cols]) * zfr
                    ob = obase[rr]
                    rvec = jnp.full((LANES,), r, jnp.int32)
                    og_e = plsc.load_gather(og_b, [rvec, evens + ob])
                    og_o = plsc.load_gather(og_b, [rvec, odds + ob])
                    u_e = plsc.load_gather(u_slab, [evens, cols])
                    u_o = plsc.load_gather(u_slab, [odds, cols])
                    acc_e = (u_e + og_e) * zfr
                    acc_o = (u_o + og_o) * zfr
                    for l in range(N_LAT):
                        ul = (uv0 if l < LANES else uv1)[l % LANES]
                        w = as_b[r, l // 8, pl.ds((l % 8) * LANES, LANES)]
                        ae = plsc.bitcast(lax.shift_left(w, 16), jnp.float32)
                        ao = plsc.bitcast(w & jnp.int32(-65536), jnp.float32)
                        acc_e = acc_e + ul * ae
                        acc_o = acc_o + ul * ao
                    plsc.store_scatter(out_slab, [evens, cols], acc_e)
                    plsc.store_scatter(out_slab, [odds, cols], acc_o)
                return carry

            lax.fori_loop(0, C // 8, octet, 0)

        start(0, as_p, og_p, sem_as_p, sem_og_p)

        def pair(i, carry):
            offp = (2 * i) * C
            offq = (2 * i + 1) * C
            start(offq, as_q, og_q, sem_as_q, sem_og_q)
            wait(as_p, og_p, sem_as_p, sem_og_p)
            compute(offp, as_p, og_p)

            @pl.when(i < NPAIR - 1)
            def _():
                start(offq + C, as_p, og_p, sem_as_p, sem_og_p)

            wait(as_q, og_q, sem_as_q, sem_og_q)
            compute(offq, as_q, og_q)
            return carry

        lax.fori_loop(0, NPAIR, pair, 0)
        pltpu.sync_copy(out_slab, out_t_hbm.at[:, pl.ds(base, RPW)])

    return k


def kernel(u, sample_index, amat_sample, offsets):
    B = u.shape[0]
    n_sample = amat_sample.shape[0]
    si = sample_index.squeeze() if sample_index.ndim > 1 else sample_index
    si = si.astype(jnp.int32)
    amat_t = amat_sample.transpose(1, 2, 0)
    offs4 = offsets.reshape(n_sample // 4, 128)
    u_t = u.T
    half = (n_sample // (2 * SB)) * SB
    sizes = [half, n_sample - half]
    lows = [0, half]
    out_t = None
    for lo, size in zip(lows, sizes):
        packed = _build_pack(size, lo // SB)(amat_t)
        part = _build_sc(B, n_sample, lo, size)(u_t, si, packed, offs4)
        out_t = part if out_t is None else out_t + part
    return out_t.T


# speedup vs baseline: 2.3391x; 1.0203x over previous
import functools

import jax
import jax.numpy as jnp
from jax import lax
from jax.experimental import pallas as pl
from jax.experimental.pallas import tpu as pltpu
from jax.experimental.pallas import tpu_sc as plsc

N_LAT = 32
N_OUT = 32
LANES = 16
SB = 512


def _pack_body(in_ref, out_ref):
    x = in_ref[...]
    u16 = jax.lax.bitcast_convert_type(
        x.astype(jnp.bfloat16), jnp.uint16).astype(jnp.int32)
    r = u16.reshape(N_LAT, LANES, 2, SB)
    w = r[:, :, 0, :] | (r[:, :, 1, :] << 16)
    wt = jnp.transpose(w.reshape(N_LAT * LANES, SB), (1, 0))
    out_ref[...] = wt.reshape(SB, 4, 128)


def _build_pack(N_SAMPLE):
    return pl.pallas_call(
        _pack_body,
        grid=(pl.cdiv(N_SAMPLE, SB),),
        in_specs=[pl.BlockSpec((N_LAT, N_OUT, SB), lambda i: (0, 0, i))],
        out_specs=pl.BlockSpec((SB, 4, 128), lambda i: (i, 0, 0)),
        out_shape=jax.ShapeDtypeStruct((N_SAMPLE, 4, 128), jnp.int32),
    )


def _build_sc(B, N_SAMPLE):
    info = plsc.get_sparse_core_info()
    NC, NS = info.num_cores, info.num_subcores
    NW = NC * NS
    assert B % NW == 0 and N_SAMPLE % 4 == 0
    RPW = B // NW
    C = 32
    NPAIR = RPW // (2 * C)

    mesh = plsc.VectorSubcoreMesh(core_axis_name="c", subcore_axis_name="s")

    @functools.partial(
        pl.kernel,
        mesh=mesh,
        out_type=jax.ShapeDtypeStruct((N_OUT, B), jnp.float32),
        compiler_params=pltpu.CompilerParams(needs_layout_passes=False),
        scratch_types=[
            pltpu.VMEM((RPW,), jnp.int32),
            pltpu.VMEM((RPW,), jnp.int32),
            pltpu.VMEM((C, 4, 128), jnp.int32),
            pltpu.VMEM((C, 4, 128), jnp.int32),
            pltpu.VMEM((C, 128), jnp.float32),
            pltpu.VMEM((C, 128), jnp.float32),
            pltpu.VMEM((N_LAT, RPW), jnp.float32),
            pltpu.VMEM((N_OUT, RPW), jnp.float32),
            pltpu.SemaphoreType.DMA,
            pltpu.SemaphoreType.DMA,
            pltpu.SemaphoreType.DMA,
            pltpu.SemaphoreType.DMA,
        ],
    )
    def k(u_t_hbm, si_hbm, amat_hbm, offs4_hbm, out_t_hbm,
          idx_v, idx4_v, as_p, as_q, og_p, og_q, u_slab, out_slab,
          sem_as_p, sem_as_q, sem_og_p, sem_og_q):
        wid = lax.axis_index("s") * NC + lax.axis_index("c")
        base = wid * RPW
        pltpu.sync_copy(si_hbm.at[pl.ds(base, RPW)], idx_v)
        pltpu.sync_copy(u_t_hbm.at[:, pl.ds(base, RPW)], u_slab)

        def shift(g, carry):
            sl = pl.ds(g * LANES, LANES)
            idx4_v[sl] = lax.shift_right_logical(idx_v[sl], 2)
            return carry

        lax.fori_loop(0, RPW // LANES, shift, 0)

        def start(off, as_b, og_b, sem_a, sem_o):
            pltpu.async_copy(amat_hbm.at[idx_v.at[pl.ds(off, C)]], as_b, sem_a)
            pltpu.async_copy(offs4_hbm.at[idx4_v.at[pl.ds(off, C)]], og_b, sem_o)

        def wait(as_b, og_b, sem_a, sem_o):
            pltpu.make_async_copy(amat_hbm.at[idx_v.at[pl.ds(0, C)]], as_b, sem_a).wait()
            pltpu.make_async_copy(offs4_hbm.at[idx4_v.at[pl.ds(0, C)]], og_b, sem_o).wait()

        def compute(off, as_b, og_b):
            def octet(ro, carry):
                rbase = ro * 8
                sv = plsc.load_gather(
                    idx_v, [lax.iota(jnp.int32, LANES) + (off + rbase)])
                obase = (sv & 3) * N_OUT
                for rr in range(8):
                    r = rbase + rr
                    j = off + r
                    rows = lax.iota(jnp.int32, LANES)
                    evens = rows * 2
                    odds = evens + 1
                    cols = jnp.full((LANES,), j, jnp.int32)
                    uv0 = plsc.load_gather(u_slab, [rows, cols])
                    uv1 = plsc.load_gather(u_slab, [rows + LANES, cols])
                    ob = obase[rr]
                    rvec = jnp.full((LANES,), r, jnp.int32)
                    og_e = plsc.load_gather(og_b, [rvec, evens + ob])
                    og_o = plsc.load_gather(og_b, [rvec, odds + ob])
                    u_e = plsc.load_gather(u_slab, [evens, cols])
                    u_o = plsc.load_gather(u_slab, [odds, cols])
                    acc_e = u_e + og_e
                    acc_o = u_o + og_o
                    for l in range(N_LAT):
                        ul = (uv0 if l < LANES else uv1)[l % LANES]
                        w = as_b[r, l // 8, pl.ds((l % 8) * LANES, LANES)]
                        ae = plsc.bitcast(lax.shift_left(w, 16), jnp.float32)
                        ao = plsc.bitcast(w & jnp.int32(-65536), jnp.float32)
                        acc_e = acc_e + ul * ae
                        acc_o = acc_o + ul * ao
                    plsc.store_scatter(out_slab, [evens, cols], acc_e)
                    plsc.store_scatter(out_slab, [odds, cols], acc_o)
                return carry

            lax.fori_loop(0, C // 8, octet, 0)

        start(0, as_p, og_p, sem_as_p, sem_og_p)

        def pair(i, carry):
            offp = (2 * i) * C
            offq = (2 * i + 1) * C
            start(offq, as_q, og_q, sem_as_q, sem_og_q)
            wait(as_p, og_p, sem_as_p, sem_og_p)
            compute(offp, as_p, og_p)

            @pl.when(i < NPAIR - 1)
            def _():
                start(offq + C, as_p, og_p, sem_as_p, sem_og_p)

            wait(as_q, og_q, sem_as_q, sem_og_q)
            compute(offq, as_q, og_q)
            return carry

        lax.fori_loop(0, NPAIR, pair, 0)
        pltpu.sync_copy(out_slab, out_t_hbm.at[:, pl.ds(base, RPW)])

    return k


def kernel(u, sample_index, amat_sample, offsets):
    B = u.shape[0]
    n_sample = amat_sample.shape[0]
    si = sample_index.squeeze() if sample_index.ndim > 1 else sample_index
    amat_t = amat_sample.transpose(1, 2, 0)
    amat_packed = _build_pack(n_sample)(amat_t)
    offs4 = offsets.reshape(n_sample // 4, 128)
    k = _build_sc(B, n_sample)
    out_t = k(u.T, si.astype(jnp.int32), amat_packed, offs4)
    return out_t.T


# speedup vs baseline: 2.6853x; 1.1480x over previous
import functools

import jax
import jax.numpy as jnp
from jax import lax
from jax.experimental import pallas as pl
from jax.experimental.pallas import tpu as pltpu
from jax.experimental.pallas import tpu_sc as plsc

N_LAT = 32
N_OUT = 32
LANES = 16
SB = 1024


def _pack_body(in_ref, out_ref):
    x = in_ref[...]
    u16 = jax.lax.bitcast_convert_type(
        x.astype(jnp.bfloat16), jnp.uint16).astype(jnp.int32)
    r = u16.reshape(N_LAT, LANES, 2, SB)
    w = r[:, :, 0, :] | (r[:, :, 1, :] << 16)
    wt = jnp.transpose(w.reshape(N_LAT * LANES, SB), (1, 0))
    out_ref[...] = wt.reshape(SB, 4, 128)


def _build_pack(N_SAMPLE):
    return pl.pallas_call(
        _pack_body,
        grid=(pl.cdiv(N_SAMPLE, SB),),
        in_specs=[pl.BlockSpec((N_LAT, N_OUT, SB), lambda i: (0, 0, i))],
        out_specs=pl.BlockSpec((SB, 4, 128), lambda i: (i, 0, 0)),
        out_shape=jax.ShapeDtypeStruct((N_SAMPLE, 4, 128), jnp.int32),
    )


def _build_sc(B, N_SAMPLE):
    info = plsc.get_sparse_core_info()
    NC, NS = info.num_cores, info.num_subcores
    NW = NC * NS
    assert B % NW == 0 and N_SAMPLE % 4 == 0
    RPW = B // NW
    C = 32
    NPAIR = RPW // (2 * C)

    mesh = plsc.VectorSubcoreMesh(core_axis_name="c", subcore_axis_name="s")

    @functools.partial(
        pl.kernel,
        mesh=mesh,
        out_type=jax.ShapeDtypeStruct((N_OUT, B), jnp.float32),
        compiler_params=pltpu.CompilerParams(needs_layout_passes=False),
        scratch_types=[
            pltpu.VMEM((RPW,), jnp.int32),
            pltpu.VMEM((RPW,), jnp.int32),
            pltpu.VMEM((C, 4, 128), jnp.int32),
            pltpu.VMEM((C, 4, 128), jnp.int32),
            pltpu.VMEM((C, 128), jnp.float32),
            pltpu.VMEM((C, 128), jnp.float32),
            pltpu.VMEM((N_LAT, RPW), jnp.float32),
            pltpu.VMEM((N_OUT, RPW), jnp.float32),
            pltpu.SemaphoreType.DMA,
            pltpu.SemaphoreType.DMA,
            pltpu.SemaphoreType.DMA,
            pltpu.SemaphoreType.DMA,
        ],
    )
    def k(u_t_hbm, si_hbm, amat_hbm, offs4_hbm, out_t_hbm,
          idx_v, idx4_v, as_p, as_q, og_p, og_q, u_slab, out_slab,
          sem_as_p, sem_as_q, sem_og_p, sem_og_q):
        wid = lax.axis_index("s") * NC + lax.axis_index("c")
        base = wid * RPW
        pltpu.sync_copy(si_hbm.at[pl.ds(base, RPW)], idx_v)
        pltpu.sync_copy(u_t_hbm.at[:, pl.ds(base, RPW)], u_slab)

        def shift(g, carry):
            sl = pl.ds(g * LANES, LANES)
            idx4_v[sl] = lax.shift_right_logical(idx_v[sl], 2)
            return carry

        lax.fori_loop(0, RPW // LANES, shift, 0)

        def start(off, as_b, og_b, sem_a, sem_o):
            pltpu.async_copy(amat_hbm.at[idx_v.at[pl.ds(off, C)]], as_b, sem_a)
            pltpu.async_copy(offs4_hbm.at[idx4_v.at[pl.ds(off, C)]], og_b, sem_o)

        def wait(as_b, og_b, sem_a, sem_o):
            pltpu.make_async_copy(amat_hbm.at[idx_v.at[pl.ds(0, C)]], as_b, sem_a).wait()
            pltpu.make_async_copy(offs4_hbm.at[idx4_v.at[pl.ds(0, C)]], og_b, sem_o).wait()

        def compute(off, as_b, og_b):
            def octet(ro, carry):
                rbase = ro * 8
                sv = plsc.load_gather(
                    idx_v, [lax.iota(jnp.int32, LANES) + (off + rbase)])
                obase = (sv & 3) * N_OUT
                for rr in range(8):
                    r = rbase + rr
                    j = off + r
                    rows = lax.iota(jnp.int32, LANES)
                    evens = rows * 2
                    odds = evens + 1
                    cols = jnp.full((LANES,), j, jnp.int32)
                    uv0 = plsc.load_gather(u_slab, [rows, cols])
                    uv1 = plsc.load_gather(u_slab, [rows + LANES, cols])
                    ob = obase[rr]
                    rvec = jnp.full((LANES,), r, jnp.int32)
                    og_e = plsc.load_gather(og_b, [rvec, evens + ob])
                    og_o = plsc.load_gather(og_b, [rvec, odds + ob])
                    u_e = plsc.load_gather(u_slab, [evens, cols])
                    u_o = plsc.load_gather(u_slab, [odds, cols])
                    acc_e = u_e + og_e
                    acc_o = u_o + og_o
                    for l in range(N_LAT):
                        ul = (uv0 if l < LANES else uv1)[l % LANES]
                        w = as_b[r, l // 8, pl.ds((l % 8) * LANES, LANES)]
                        ae = plsc.bitcast(lax.shift_left(w, 16), jnp.float32)
                        ao = plsc.bitcast(w & jnp.int32(-65536), jnp.float32)
                        acc_e = acc_e + ul * ae
                        acc_o = acc_o + ul * ao
                    plsc.store_scatter(out_slab, [evens, cols], acc_e)
                    plsc.store_scatter(out_slab, [odds, cols], acc_o)
                return carry

            lax.fori_loop(0, C // 8, octet, 0)

        start(0, as_p, og_p, sem_as_p, sem_og_p)

        def pair(i, carry):
            offp = (2 * i) * C
            offq = (2 * i + 1) * C
            start(offq, as_q, og_q, sem_as_q, sem_og_q)
            wait(as_p, og_p, sem_as_p, sem_og_p)
            compute(offp, as_p, og_p)

            @pl.when(i < NPAIR - 1)
            def _():
                start(offq + C, as_p, og_p, sem_as_p, sem_og_p)

            wait(as_q, og_q, sem_as_q, sem_og_q)
            compute(offq, as_q, og_q)
            return carry

        lax.fori_loop(0, NPAIR, pair, 0)
        pltpu.sync_copy(out_slab, out_t_hbm.at[:, pl.ds(base, RPW)])

    return k


def kernel(u, sample_index, amat_sample, offsets):
    B = u.shape[0]
    n_sample = amat_sample.shape[0]
    si = sample_index.squeeze() if sample_index.ndim > 1 else sample_index
    amat_t = amat_sample.transpose(1, 2, 0)
    amat_packed = _build_pack(n_sample)(amat_t)
    offs4 = offsets.reshape(n_sample // 4, 128)
    k = _build_sc(B, n_sample)
    out_t = k(u.T, si.astype(jnp.int32), amat_packed, offs4)
    return out_t.T


# speedup vs baseline: 2.9066x; 1.0824x over previous
import functools

import jax
import jax.numpy as jnp
from jax import lax
from jax.experimental import pallas as pl
from jax.experimental.pallas import tpu as pltpu
from jax.experimental.pallas import tpu_sc as plsc

N_LAT = 32
N_OUT = 32
LANES = 16
SB = 2048


def _pack_body(in_ref, out_ref):
    x = in_ref[...]
    u16 = jax.lax.bitcast_convert_type(
        x.astype(jnp.bfloat16), jnp.uint16).astype(jnp.int32)
    r = u16.reshape(N_LAT, LANES, 2, SB)
    w = r[:, :, 0, :] | (r[:, :, 1, :] << 16)
    wt = jnp.transpose(w.reshape(N_LAT * LANES, SB), (1, 0))
    out_ref[...] = wt.reshape(SB, 4, 128)


def _build_pack(N_SAMPLE):
    return pl.pallas_call(
        _pack_body,
        grid=(pl.cdiv(N_SAMPLE, SB),),
        in_specs=[pl.BlockSpec((N_LAT, N_OUT, SB), lambda i: (0, 0, i))],
        out_specs=pl.BlockSpec((SB, 4, 128), lambda i: (i, 0, 0)),
        out_shape=jax.ShapeDtypeStruct((N_SAMPLE, 4, 128), jnp.int32),
    )


def _build_sc(B, N_SAMPLE):
    info = plsc.get_sparse_core_info()
    NC, NS = info.num_cores, info.num_subcores
    NW = NC * NS
    assert B % NW == 0 and N_SAMPLE % 4 == 0
    RPW = B // NW
    C = 32
    NPAIR = RPW // (2 * C)

    mesh = plsc.VectorSubcoreMesh(core_axis_name="c", subcore_axis_name="s")

    @functools.partial(
        pl.kernel,
        mesh=mesh,
        out_type=jax.ShapeDtypeStruct((N_OUT, B), jnp.float32),
        compiler_params=pltpu.CompilerParams(needs_layout_passes=False),
        scratch_types=[
            pltpu.VMEM((RPW,), jnp.int32),
            pltpu.VMEM((RPW,), jnp.int32),
            pltpu.VMEM((C, 4, 128), jnp.int32),
            pltpu.VMEM((C, 4, 128), jnp.int32),
            pltpu.VMEM((C, 128), jnp.float32),
            pltpu.VMEM((C, 128), jnp.float32),
            pltpu.VMEM((N_LAT, RPW), jnp.float32),
            pltpu.VMEM((N_OUT, RPW), jnp.float32),
            pltpu.SemaphoreType.DMA,
            pltpu.SemaphoreType.DMA,
            pltpu.SemaphoreType.DMA,
            pltpu.SemaphoreType.DMA,
        ],
    )
    def k(u_t_hbm, si_hbm, amat_hbm, offs4_hbm, out_t_hbm,
          idx_v, idx4_v, as_p, as_q, og_p, og_q, u_slab, out_slab,
          sem_as_p, sem_as_q, sem_og_p, sem_og_q):
        wid = lax.axis_index("s") * NC + lax.axis_index("c")
        base = wid * RPW
        pltpu.sync_copy(si_hbm.at[pl.ds(base, RPW)], idx_v)
        pltpu.sync_copy(u_t_hbm.at[:, pl.ds(base, RPW)], u_slab)

        def shift(g, carry):
            sl = pl.ds(g * LANES, LANES)
            idx4_v[sl] = lax.shift_right_logical(idx_v[sl], 2)
            return carry

        lax.fori_loop(0, RPW // LANES, shift, 0)

        def start(off, as_b, og_b, sem_a, sem_o):
            pltpu.async_copy(amat_hbm.at[idx_v.at[pl.ds(off, C)]], as_b, sem_a)
            pltpu.async_copy(offs4_hbm.at[idx4_v.at[pl.ds(off, C)]], og_b, sem_o)

        def wait(as_b, og_b, sem_a, sem_o):
            pltpu.make_async_copy(amat_hbm.at[idx_v.at[pl.ds(0, C)]], as_b, sem_a).wait()
            pltpu.make_async_copy(offs4_hbm.at[idx4_v.at[pl.ds(0, C)]], og_b, sem_o).wait()

        def compute(off, as_b, og_b):
            def octet(ro, carry):
                rbase = ro * 8
                sv = plsc.load_gather(
                    idx_v, [lax.iota(jnp.int32, LANES) + (off + rbase)])
                obase = (sv & 3) * N_OUT
                for rr in range(8):
                    r = rbase + rr
                    j = off + r
                    rows = lax.iota(jnp.int32, LANES)
                    evens = rows * 2
                    odds = evens + 1
                    cols = jnp.full((LANES,), j, jnp.int32)
                    uv0 = plsc.load_gather(u_slab, [rows, cols])
                    uv1 = plsc.load_gather(u_slab, [rows + LANES, cols])
                    ob = obase[rr]
                    rvec = jnp.full((LANES,), r, jnp.int32)
                    og_e = plsc.load_gather(og_b, [rvec, evens + ob])
                    og_o = plsc.load_gather(og_b, [rvec, odds + ob])
                    u_e = plsc.load_gather(u_slab, [evens, cols])
                    u_o = plsc.load_gather(u_slab, [odds, cols])
                    acc_e = u_e + og_e
                    acc_o = u_o + og_o
                    for l in range(N_LAT):
                        ul = (uv0 if l < LANES else uv1)[l % LANES]
                        w = as_b[r, l // 8, pl.ds((l % 8) * LANES, LANES)]
                        ae = plsc.bitcast(lax.shift_left(w, 16), jnp.float32)
                        ao = plsc.bitcast(w & jnp.int32(-65536), jnp.float32)
                        acc_e = acc_e + ul * ae
                        acc_o = acc_o + ul * ao
                    plsc.store_scatter(out_slab, [evens, cols], acc_e)
                    plsc.store_scatter(out_slab, [odds, cols], acc_o)
                return carry

            lax.fori_loop(0, C // 8, octet, 0)

        start(0, as_p, og_p, sem_as_p, sem_og_p)

        def pair(i, carry):
            offp = (2 * i) * C
            offq = (2 * i + 1) * C
            start(offq, as_q, og_q, sem_as_q, sem_og_q)
            wait(as_p, og_p, sem_as_p, sem_og_p)
            compute(offp, as_p, og_p)

            @pl.when(i < NPAIR - 1)
            def _():
                start(offq + C, as_p, og_p, sem_as_p, sem_og_p)

            wait(as_q, og_q, sem_as_q, sem_og_q)
            compute(offq, as_q, og_q)
            return carry

        lax.fori_loop(0, NPAIR, pair, 0)
        pltpu.sync_copy(out_slab, out_t_hbm.at[:, pl.ds(base, RPW)])

    return k


def kernel(u, sample_index, amat_sample, offsets):
    B = u.shape[0]
    n_sample = amat_sample.shape[0]
    si = sample_index.squeeze() if sample_index.ndim > 1 else sample_index
    amat_t = amat_sample.transpose(1, 2, 0)
    amat_packed = _build_pack(n_sample)(amat_t)
    offs4 = offsets.reshape(n_sample // 4, 128)
    k = _build_sc(B, n_sample)
    out_t = k(u.T, si.astype(jnp.int32), amat_packed, offs4)
    return out_t.T
